# Initial kernel scaffold; baseline (speedup 1.0000x reference)
import jax
import jax.numpy as jnp
from jax.experimental import pallas as pl


def kernel(wordid, mask, emb, W_iou, U_iou, b_iou, U_f_W, U_f_b, lin_W, lin_b):
    raise NotImplementedError("write your pallas kernel here")


# speedup vs baseline: 5.9392x; 5.9392x over previous
import functools

import jax
import jax.numpy as jnp
from jax import lax
from jax.experimental import pallas as pl
from jax.experimental.pallas import tpu as pltpu
from jax.experimental.pallas import tpu_sc as plsc

H = 128
D = 17
N = 2**D - 1
LEAF = 2 ** (D - 1)

NC, NS = 2, 16
NW = NC * NS
ROWS_W = LEAF // NW
CHUNK = 128
NCHUNK = ROWS_W // CHUNK


def _gather_body(emb_hbm, idx_hbm, out_hbm, idx_v, rows0, rows1, sem0, sem1):
    wid = lax.axis_index("s") * NC + lax.axis_index("c")
    pltpu.sync_copy(idx_hbm.at[pl.ds(wid * NCHUNK, NCHUNK)], idx_v)
    bufs = (rows0, rows1)
    sems = (sem0, sem1)
    pltpu.async_copy(emb_hbm.at[idx_v.at[0]], bufs[0], sems[0])
    for c in range(NCHUNK):
        if c + 1 < NCHUNK:
            pltpu.async_copy(
                emb_hbm.at[idx_v.at[c + 1]], bufs[(c + 1) % 2], sems[(c + 1) % 2]
            )
        pltpu.make_async_copy(
            emb_hbm.at[idx_v.at[c]], bufs[c % 2], sems[c % 2]
        ).wait()
        pltpu.sync_copy(
            bufs[c % 2], out_hbm.at[pl.ds(wid * ROWS_W + c * CHUNK, CHUNK)]
        )


def _make_sc_gather(interpret=False):
    return pl.kernel(
        _gather_body,
        out_type=jax.ShapeDtypeStruct((LEAF, H), jnp.float32),
        mesh=plsc.VectorSubcoreMesh(
            core_axis_name="c", subcore_axis_name="s",
            num_cores=NC, num_subcores=NS,
        ),
        scratch_types=[
            pltpu.VMEM((NCHUNK, CHUNK), jnp.int32),
            pltpu.VMEM((CHUNK, H), jnp.float32),
            pltpu.VMEM((CHUNK, H), jnp.float32),
            pltpu.SemaphoreType.DMA,
            pltpu.SemaphoreType.DMA,
        ],
        interpret=interpret,
    )


def _leaf_body(e_ref, m_ref, w_ref, b_ref, lw_ref, lb_ref,
               h_ref, c_ref, mh_ref, lg_ref):
    iou = (
        jnp.dot(e_ref[...], w_ref[...], preferred_element_type=jnp.float32)
        * m_ref[...]
        + b_ref[...]
    )
    i = jax.nn.sigmoid(iou[:, :H])
    o = jax.nn.sigmoid(iou[:, H:2 * H])
    u = jnp.tanh(iou[:, 2 * H:])
    c = i * u
    h = o * jnp.tanh(c)
    mh = jnp.maximum(h, 0.0)
    h_ref[...] = h
    c_ref[...] = c
    mh_ref[...] = mh
    lg_ref[...] = (
        jnp.dot(h + mh, lw_ref[...], preferred_element_type=jnp.float32)
        + lb_ref[...]
    )


@functools.cache
def _make_leaf_call(ncls, interpret=False):
    R = 512
    grid = (LEAF // R,)
    return pl.pallas_call(
        _leaf_body,
        grid=grid,
        in_specs=[
            pl.BlockSpec((R, H), lambda g: (g, 0)),
            pl.BlockSpec((R, 1), lambda g: (g, 0)),
            pl.BlockSpec((H, 3 * H), lambda g: (0, 0)),
            pl.BlockSpec((1, 3 * H), lambda g: (0, 0)),
            pl.BlockSpec((H, ncls), lambda g: (0, 0)),
            pl.BlockSpec((1, ncls), lambda g: (0, 0)),
        ],
        out_specs=[
            pl.BlockSpec((R, H), lambda g: (g, 0)),
            pl.BlockSpec((R, H), lambda g: (g, 0)),
            pl.BlockSpec((R, H), lambda g: (g, 0)),
            pl.BlockSpec((R, ncls), lambda g: (g, 0)),
        ],
        out_shape=[
            jax.ShapeDtypeStruct((LEAF, H), jnp.float32),
            jax.ShapeDtypeStruct((LEAF, H), jnp.float32),
            jax.ShapeDtypeStruct((LEAF, H), jnp.float32),
            jax.ShapeDtypeStruct((LEAF, ncls), jnp.float32),
        ],
        interpret=interpret,
    )


def _level_body(hc_ref, cc_ref, mc_ref, w_ref, b_ref, lw_ref, lb_ref,
                h_ref, c_ref, mh_ref, lg_ref):
    g = (
        jnp.dot(hc_ref[...], w_ref[...], preferred_element_type=jnp.float32)
        + b_ref[...]
    )
    f = jax.nn.sigmoid(g[:, :2 * H])
    cc = cc_ref[...]
    c_red = f[:, :H] * cc[:, :H] + f[:, H:] * cc[:, H:]
    i = jax.nn.sigmoid(g[:, 2 * H:3 * H])
    o = jax.nn.sigmoid(g[:, 3 * H:4 * H])
    u = jnp.tanh(g[:, 4 * H:])
    c = i * u + c_red
    h = o * jnp.tanh(c)
    mc = mc_ref[...]
    mh = jnp.maximum(h, jnp.maximum(mc[:, :H], mc[:, H:]))
    h_ref[...] = h
    c_ref[...] = c
    mh_ref[...] = mh
    lg_ref[...] = (
        jnp.dot(h + mh, lw_ref[...], preferred_element_type=jnp.float32)
        + lb_ref[...]
    )


@functools.cache
def _make_level_call(s, ncls, interpret=False):
    R = min(s, 512)
    grid = (s // R,)
    return pl.pallas_call(
        _level_body,
        grid=grid,
        in_specs=[
            pl.BlockSpec((R, 2 * H), lambda g: (g, 0)),
            pl.BlockSpec((R, 2 * H), lambda g: (g, 0)),
            pl.BlockSpec((R, 2 * H), lambda g: (g, 0)),
            pl.BlockSpec((2 * H, 5 * H), lambda g: (0, 0)),
            pl.BlockSpec((1, 5 * H), lambda g: (0, 0)),
            pl.BlockSpec((H, ncls), lambda g: (0, 0)),
            pl.BlockSpec((1, ncls), lambda g: (0, 0)),
        ],
        out_specs=[
            pl.BlockSpec((R, H), lambda g: (g, 0)),
            pl.BlockSpec((R, H), lambda g: (g, 0)),
            pl.BlockSpec((R, H), lambda g: (g, 0)),
            pl.BlockSpec((R, ncls), lambda g: (g, 0)),
        ],
        out_shape=[
            jax.ShapeDtypeStruct((s, H), jnp.float32),
            jax.ShapeDtypeStruct((s, H), jnp.float32),
            jax.ShapeDtypeStruct((s, H), jnp.float32),
            jax.ShapeDtypeStruct((s, ncls), jnp.float32),
        ],
        interpret=interpret,
    )


def _tree_lstm(leaf_emb, maskf, w_iou_t, b_iou, w_all, b_all, lin_W, lin_b,
               interpret=False):
    ncls = lin_W.shape[0]
    lw = lin_W.T
    lb = lin_b.reshape(1, ncls)
    h, c, mh, lg = _make_leaf_call(ncls, interpret)(
        leaf_emb, maskf, w_iou_t, b_iou, lw, lb
    )
    parts = [lg]
    for l in range(D - 2, -1, -1):
        s = 2 ** l
        hc = h.reshape(s, 2 * H)
        cc = c.reshape(s, 2 * H)
        mc = mh.reshape(s, 2 * H)
        h, c, mh, lg = _make_level_call(s, ncls, interpret)(
            hc, cc, mc, w_all, b_all, lw, lb
        )
        parts.append(lg)
    return jnp.concatenate(parts[::-1], axis=0)


def kernel(wordid, mask, emb, W_iou, U_iou, b_iou, U_f_W, U_f_b, lin_W, lin_b):
    leaf_wid = (wordid[LEAF - 1:] * mask[LEAF - 1:]).astype(jnp.int32)
    leaf_wid = leaf_wid.reshape(NW * NCHUNK, CHUNK)
    maskf = mask[LEAF - 1:].astype(jnp.float32).reshape(LEAF, 1)
    leaf_emb = _make_sc_gather()(emb, leaf_wid)
    w_all = jnp.concatenate([U_f_W, U_iou], axis=0).T
    b_all = jnp.concatenate([U_f_b, b_iou[0]]).reshape(1, 5 * H)
    return _tree_lstm(leaf_emb, maskf, W_iou.T, b_iou, w_all, b_all,
                      lin_W, lin_b)


# speedup vs baseline: 7.6688x; 1.2912x over previous
import functools

import jax
import jax.numpy as jnp
from jax import lax
from jax.experimental import pallas as pl
from jax.experimental.pallas import tpu as pltpu
from jax.experimental.pallas import tpu_sc as plsc

H = 128
D = 17
N = 2**D - 1
LEAF = 2 ** (D - 1)

NC, NS = 2, 16
NW = NC * NS
ROWS_W = LEAF // NW
BIG = 512

SUB = 2048
NSUB = LEAF // SUB
TOPL = 5


def _gather_body(emb_hbm, idx_hbm, out_hbm, idx_v, rows, sem):
    wid = lax.axis_index("s") * NC + lax.axis_index("c")
    pltpu.sync_copy(idx_hbm.at[pl.ds(wid * ROWS_W, ROWS_W)], idx_v)
    for c in range(ROWS_W // BIG):
        pltpu.async_copy(
            emb_hbm.at[idx_v.at[pl.ds(c * BIG, BIG)]], rows, sem
        ).wait()
        pltpu.sync_copy(
            rows, out_hbm.at[pl.ds(wid * ROWS_W + c * BIG, BIG)]
        )


def _make_sc_gather(interpret=False):
    return pl.kernel(
        _gather_body,
        out_type=jax.ShapeDtypeStruct((LEAF, H), jnp.float32),
        mesh=plsc.VectorSubcoreMesh(
            core_axis_name="c", subcore_axis_name="s",
            num_cores=NC, num_subcores=NS,
        ),
        scratch_types=[
            pltpu.VMEM((ROWS_W,), jnp.int32),
            pltpu.VMEM((BIG, H), jnp.float32),
            pltpu.SemaphoreType.DMA,
        ],
        interpret=interpret,
    )


def _gates(iou, c_red):
    i = jax.nn.sigmoid(iou[:, :H])
    o = jax.nn.sigmoid(iou[:, H:2 * H])
    u = jnp.tanh(iou[:, 2 * H:])
    c = i * u + c_red
    h = o * jnp.tanh(c)
    return h, c


def _subtree_body(e_ref, m_ref, w_ref, b_ref, wa_ref, ba_ref, lw_ref, lb_ref,
                  *out_refs):
    lw = lw_ref[...]
    lb = lb_ref[...]
    iou = (
        jnp.dot(e_ref[...], w_ref[...], preferred_element_type=jnp.float32)
        * m_ref[...]
        + b_ref[...]
    )
    h, c = _gates(iou, 0.0)
    mh = jnp.maximum(h, 0.0)
    lg0 = jnp.dot(h + mh, lw, preferred_element_type=jnp.float32) + lb
    out_refs[0][...] = lg0.reshape(out_refs[0].shape)
    wa = wa_ref[...]
    ba = ba_ref[...]
    s = SUB
    for k in range(1, D - TOPL):
        s //= 2
        hc = h.reshape(s, 2 * H)
        cc = c.reshape(s, 2 * H)
        mc = mh.reshape(s, 2 * H)
        g = jnp.dot(hc, wa, preferred_element_type=jnp.float32) + ba
        f = jax.nn.sigmoid(g[:, :2 * H])
        c_red = f[:, :H] * cc[:, :H] + f[:, H:] * cc[:, H:]
        h, c = _gates(g[:, 2 * H:], c_red)
        mh = jnp.maximum(h, jnp.maximum(mc[:, :H], mc[:, H:]))
        lg = jnp.dot(h + mh, lw, preferred_element_type=jnp.float32) + lb
        out_refs[k][...] = lg.reshape(out_refs[k].shape)
    out_refs[D - TOPL][...] = h.reshape(1, 1, H)
    out_refs[D - TOPL + 1][...] = c.reshape(1, 1, H)
    out_refs[D - TOPL + 2][...] = mh.reshape(1, 1, H)


@functools.cache
def _make_subtree_call(ncls, interpret=False):
    lg_specs, lg_shapes = [], []
    for l in range(D - 1, TOPL - 1, -1):
        bs = 2 ** (l - TOPL)
        lg_specs.append(pl.BlockSpec((1, bs, ncls), lambda g: (g, 0, 0)))
        lg_shapes.append(jax.ShapeDtypeStruct((NSUB, bs, ncls), jnp.float32))
    fr_spec = pl.BlockSpec((1, 1, H), lambda g: (g, 0, 0))
    fr_shape = jax.ShapeDtypeStruct((NSUB, 1, H), jnp.float32)
    return pl.pallas_call(
        _subtree_body,
        grid=(NSUB,),
        in_specs=[
            pl.BlockSpec((SUB, H), lambda g: (g, 0)),
            pl.BlockSpec((SUB, 1), lambda g: (g, 0)),
            pl.BlockSpec((H, 3 * H), lambda g: (0, 0)),
            pl.BlockSpec((1, 3 * H), lambda g: (0, 0)),
            pl.BlockSpec((2 * H, 5 * H), lambda g: (0, 0)),
            pl.BlockSpec((1, 5 * H), lambda g: (0, 0)),
            pl.BlockSpec((H, ncls), lambda g: (0, 0)),
            pl.BlockSpec((1, ncls), lambda g: (0, 0)),
        ],
        out_specs=[*lg_specs, fr_spec, fr_spec, fr_spec],
        out_shape=[*lg_shapes, fr_shape, fr_shape, fr_shape],
        interpret=interpret,
    )


def _top_body(hc_ref, cc_ref, mc_ref, wa_ref, ba_ref, lw_ref, lb_ref, lg_ref):
    lw = lw_ref[...]
    lb = lb_ref[...]
    wa = wa_ref[...]
    ba = ba_ref[...]
    hc, cc, mc = hc_ref[...], cc_ref[...], mc_ref[...]
    for l in range(TOPL - 1, -1, -1):
        s = 2 ** l
        g = jnp.dot(hc, wa, preferred_element_type=jnp.float32) + ba
        f = jax.nn.sigmoid(g[:, :2 * H])
        c_red = f[:, :H] * cc[:, :H] + f[:, H:] * cc[:, H:]
        h, c = _gates(g[:, 2 * H:], c_red)
        mh = jnp.maximum(h, jnp.maximum(mc[:, :H], mc[:, H:]))
        lg_ref[pl.ds(s - 1, s), :] = (
            jnp.dot(h + mh, lw, preferred_element_type=jnp.float32) + lb
        )
        if l > 0:
            hc = h.reshape(s // 2, 2 * H)
            cc = c.reshape(s // 2, 2 * H)
            mc = mh.reshape(s // 2, 2 * H)


@functools.cache
def _make_top_call(ncls, interpret=False):
    s5 = 2 ** TOPL
    return pl.pallas_call(
        _top_body,
        out_shape=jax.ShapeDtypeStruct((s5 - 1, ncls), jnp.float32),
        interpret=interpret,
    )


def _tree_lstm(leaf_emb, maskf, w_iou_t, b_iou, w_all, b_all, lin_W, lin_b,
               interpret=False):
    ncls = lin_W.shape[0]
    lw = lin_W.T
    lb = lin_b.reshape(1, ncls)
    outs = _make_subtree_call(ncls, interpret)(
        leaf_emb, maskf, w_iou_t, b_iou, w_all, b_all, lw, lb
    )
    lgs = [o.reshape(-1, ncls) for o in outs[:D - TOPL]]
    h5, c5, mh5 = outs[D - TOPL:]
    s5 = 2 ** TOPL
    top_lg = _make_top_call(ncls, interpret)(
        h5.reshape(s5 // 2, 2 * H), c5.reshape(s5 // 2, 2 * H),
        mh5.reshape(s5 // 2, 2 * H), w_all, b_all, lw, lb
    )
    return jnp.concatenate([top_lg, *lgs[::-1]], axis=0)


def kernel(wordid, mask, emb, W_iou, U_iou, b_iou, U_f_W, U_f_b, lin_W, lin_b):
    leaf_wid = (wordid[LEAF - 1:] * mask[LEAF - 1:]).astype(jnp.int32)
    maskf = mask[LEAF - 1:].astype(jnp.float32).reshape(LEAF, 1)
    leaf_emb = _make_sc_gather()(emb, leaf_wid)
    w_all = jnp.concatenate([U_f_W, U_iou], axis=0).T
    b_all = jnp.concatenate([U_f_b, b_iou[0]]).reshape(1, 5 * H)
    return _tree_lstm(leaf_emb, maskf, W_iou.T, b_iou, w_all, b_all,
                      lin_W, lin_b)


# speedup vs baseline: 7.7356x; 1.0087x over previous
import functools

import jax
import jax.numpy as jnp
from jax import lax
from jax.experimental import pallas as pl
from jax.experimental.pallas import tpu as pltpu
from jax.experimental.pallas import tpu_sc as plsc

H = 128
D = 17
N = 2**D - 1
LEAF = 2 ** (D - 1)

NC, NS = 2, 16
NW = NC * NS
ROWS_W = LEAF // NW
BIG = 512

SUB = 2048
NSUB = LEAF // SUB
TOPL = 5


def _gather_body(emb_hbm, idx_hbm, out_hbm, idx_v, rows, sem):
    rows_w = idx_hbm.shape[0] // NW
    wid = lax.axis_index("s") * NC + lax.axis_index("c")
    pltpu.sync_copy(idx_hbm.at[pl.ds(wid * rows_w, rows_w)], idx_v)
    for c in range(rows_w // BIG):
        pltpu.async_copy(
            emb_hbm.at[idx_v.at[pl.ds(c * BIG, BIG)]], rows, sem
        ).wait()
        pltpu.sync_copy(
            rows, out_hbm.at[pl.ds(wid * rows_w + c * BIG, BIG)]
        )


def _make_sc_gather(nrows=LEAF, interpret=False):
    return pl.kernel(
        _gather_body,
        out_type=jax.ShapeDtypeStruct((nrows, H), jnp.float32),
        mesh=plsc.VectorSubcoreMesh(
            core_axis_name="c", subcore_axis_name="s",
            num_cores=NC, num_subcores=NS,
        ),
        scratch_types=[
            pltpu.VMEM((nrows // NW,), jnp.int32),
            pltpu.VMEM((BIG, H), jnp.float32),
            pltpu.SemaphoreType.DMA,
        ],
        interpret=interpret,
    )


def _gates(iou, c_red):
    i = jax.nn.sigmoid(iou[:, :H])
    o = jax.nn.sigmoid(iou[:, H:2 * H])
    u = jnp.tanh(iou[:, 2 * H:])
    c = i * u + c_red
    h = o * jnp.tanh(c)
    return h, c


def _subtree_body(e_ref, m_ref, w_ref, b_ref, wa_ref, ba_ref, lw_ref, lb_ref,
                  *out_refs):
    lw = lw_ref[...]
    lb = lb_ref[...]
    iou = (
        jnp.dot(e_ref[...], w_ref[...], preferred_element_type=jnp.float32)
        * m_ref[...]
        + b_ref[...]
    )
    h, c = _gates(iou, 0.0)
    mh = jnp.maximum(h, 0.0)
    lg0 = jnp.dot(h + mh, lw, preferred_element_type=jnp.float32) + lb
    out_refs[0][...] = lg0.reshape(out_refs[0].shape)
    wa = wa_ref[...]
    ba = ba_ref[...]
    s = SUB
    for k in range(1, D - TOPL):
        s //= 2
        hc = h.reshape(s, 2 * H)
        cc = c.reshape(s, 2 * H)
        mc = mh.reshape(s, 2 * H)
        g = jnp.dot(hc, wa, preferred_element_type=jnp.float32) + ba
        f = jax.nn.sigmoid(g[:, :2 * H])
        c_red = f[:, :H] * cc[:, :H] + f[:, H:] * cc[:, H:]
        h, c = _gates(g[:, 2 * H:], c_red)
        mh = jnp.maximum(h, jnp.maximum(mc[:, :H], mc[:, H:]))
        lg = jnp.dot(h + mh, lw, preferred_element_type=jnp.float32) + lb
        out_refs[k][...] = lg.reshape(out_refs[k].shape)
    out_refs[D - TOPL][...] = h.reshape(1, 1, H)
    out_refs[D - TOPL + 1][...] = c.reshape(1, 1, H)
    out_refs[D - TOPL + 2][...] = mh.reshape(1, 1, H)


@functools.cache
def _make_subtree_call(ncls, nsub=NSUB, interpret=False):
    lg_specs, lg_shapes = [], []
    for l in range(D - 1, TOPL - 1, -1):
        bs = 2 ** (l - TOPL)
        lg_specs.append(pl.BlockSpec((1, bs, ncls), lambda g: (g, 0, 0)))
        lg_shapes.append(jax.ShapeDtypeStruct((nsub, bs, ncls), jnp.float32))
    fr_spec = pl.BlockSpec((1, 1, H), lambda g: (g, 0, 0))
    fr_shape = jax.ShapeDtypeStruct((nsub, 1, H), jnp.float32)
    return pl.pallas_call(
        _subtree_body,
        grid=(nsub,),
        in_specs=[
            pl.BlockSpec((SUB, H), lambda g: (g, 0)),
            pl.BlockSpec((SUB, 1), lambda g: (g, 0)),
            pl.BlockSpec((H, 3 * H), lambda g: (0, 0)),
            pl.BlockSpec((1, 3 * H), lambda g: (0, 0)),
            pl.BlockSpec((2 * H, 5 * H), lambda g: (0, 0)),
            pl.BlockSpec((1, 5 * H), lambda g: (0, 0)),
            pl.BlockSpec((H, ncls), lambda g: (0, 0)),
            pl.BlockSpec((1, ncls), lambda g: (0, 0)),
        ],
        out_specs=[*lg_specs, fr_spec, fr_spec, fr_spec],
        out_shape=[*lg_shapes, fr_shape, fr_shape, fr_shape],
        interpret=interpret,
    )


def _top_body(hc_ref, cc_ref, mc_ref, wa_ref, ba_ref, lw_ref, lb_ref, lg_ref):
    lw = lw_ref[...]
    lb = lb_ref[...]
    wa = wa_ref[...]
    ba = ba_ref[...]
    hc, cc, mc = hc_ref[...], cc_ref[...], mc_ref[...]
    for l in range(TOPL - 1, -1, -1):
        s = 2 ** l
        g = jnp.dot(hc, wa, preferred_element_type=jnp.float32) + ba
        f = jax.nn.sigmoid(g[:, :2 * H])
        c_red = f[:, :H] * cc[:, :H] + f[:, H:] * cc[:, H:]
        h, c = _gates(g[:, 2 * H:], c_red)
        mh = jnp.maximum(h, jnp.maximum(mc[:, :H], mc[:, H:]))
        lg_ref[pl.ds(s - 1, s), :] = (
            jnp.dot(h + mh, lw, preferred_element_type=jnp.float32) + lb
        )
        if l > 0:
            hc = h.reshape(s // 2, 2 * H)
            cc = c.reshape(s // 2, 2 * H)
            mc = mh.reshape(s // 2, 2 * H)


@functools.cache
def _make_top_call(ncls, interpret=False):
    s5 = 2 ** TOPL
    return pl.pallas_call(
        _top_body,
        out_shape=jax.ShapeDtypeStruct((s5 - 1, ncls), jnp.float32),
        interpret=interpret,
    )


def _tree_lstm(leaf_embs, maskfs, w_iou_t, b_iou, w_all, b_all, lin_W, lin_b,
               interpret=False):
    ncls = lin_W.shape[0]
    lw = lin_W.T
    lb = lin_b.reshape(1, ncls)
    nhalf = len(leaf_embs)
    houts = []
    for le, mf in zip(leaf_embs, maskfs):
        nsub = le.shape[0] // SUB
        houts.append(_make_subtree_call(ncls, nsub, interpret)(
            le, mf, w_iou_t, b_iou, w_all, b_all, lw, lb
        ))
    lgs = [jnp.concatenate([o.reshape(-1, ncls) for o in
                            [ho[k] for ho in houts]], axis=0)
           for k in range(D - TOPL)]
    h5 = jnp.concatenate([ho[D - TOPL].reshape(-1, H) for ho in houts], axis=0)
    c5 = jnp.concatenate([ho[D - TOPL + 1].reshape(-1, H) for ho in houts], axis=0)
    mh5 = jnp.concatenate([ho[D - TOPL + 2].reshape(-1, H) for ho in houts], axis=0)
    s5 = 2 ** TOPL
    top_lg = _make_top_call(ncls, interpret)(
        h5.reshape(s5 // 2, 2 * H), c5.reshape(s5 // 2, 2 * H),
        mh5.reshape(s5 // 2, 2 * H), w_all, b_all, lw, lb
    )
    return jnp.concatenate([top_lg, *lgs[::-1]], axis=0)


NHALF = 2


def kernel(wordid, mask, emb, W_iou, U_iou, b_iou, U_f_W, U_f_b, lin_W, lin_b):
    leaf_wid = (wordid[LEAF - 1:] * mask[LEAF - 1:]).astype(jnp.int32)
    maskf = mask[LEAF - 1:].astype(jnp.float32).reshape(LEAF, 1)
    hn = LEAF // NHALF
    leaf_embs = [_make_sc_gather(hn)(emb, leaf_wid[i * hn:(i + 1) * hn])
                 for i in range(NHALF)]
    maskfs = [maskf[i * hn:(i + 1) * hn] for i in range(NHALF)]
    w_all = jnp.concatenate([U_f_W, U_iou], axis=0).T
    b_all = jnp.concatenate([U_f_b, b_iou[0]]).reshape(1, 5 * H)
    return _tree_lstm(leaf_embs, maskfs, W_iou.T, b_iou, w_all, b_all,
                      lin_W, lin_b)


# speedup vs baseline: 20.5558x; 2.6573x over previous
import functools

import jax
import jax.numpy as jnp
from jax import lax
from jax.experimental import pallas as pl
from jax.experimental.pallas import tpu as pltpu
from jax.experimental.pallas import tpu_sc as plsc

H = 128
D = 17
N = 2**D - 1
LEAF = 2 ** (D - 1)

NC, NS = 2, 16
NW = NC * NS
ROWS_W = LEAF // NW
BIG = 512

SUB = 2048
NSUB = LEAF // SUB
TOPL = 5


CH = 256
GU = 64
SENT = CH


def _gather_body(emb_hbm, idx_hbm, msk_hbm, out_hbm,
                 idx_v, msk_v, cidx_v, smap_v, gbuf, dbuf, sem):
    rows_w = idx_hbm.shape[0] // NW
    wid = lax.axis_index("s") * NC + lax.axis_index("c")
    base = wid * rows_w
    pltpu.sync_copy(idx_hbm.at[pl.ds(base, rows_w)], idx_v)
    pltpu.sync_copy(msk_hbm.at[pl.ds(base, rows_w)], msk_v)
    for v in range(CH // 16):
        cidx_v[pl.ds(v * 16, 16)] = jnp.zeros((16,), jnp.int32)
    for k in range(H // 16):
        gbuf[SENT, pl.ds(k * 16, 16)] = jnp.zeros((16,), jnp.float32)
    for c in range(rows_w // CH):
        cbase = c * CH
        cnt = jnp.int32(0)
        for v in range(CH // 16):
            m = msk_v[pl.ds(cbase + v * 16, 16)]
            w = idx_v[pl.ds(cbase + v * 16, 16)]
            mb = m > 0
            pos = plsc.cumsum(m) + (cnt - 1)
            smap_v[pl.ds(v * 16, 16)] = jnp.where(mb, pos, SENT)
            plsc.store_scatter(cidx_v, [pos], w, mask=mb)
            cnt = cnt + jnp.sum(m)

        def gath_one(i, carry):
            pltpu.async_copy(
                emb_hbm.at[cidx_v.at[pl.ds(i * GU, GU)]],
                gbuf.at[pl.ds(i * GU, GU)], sem,
            ).wait()
            return carry

        lax.fori_loop(0, (cnt + (GU - 1)) // GU, gath_one, 0)

        def exp_vreg(vr, carry):
            v = smap_v[pl.ds(vr * 16, 16)]
            for j in range(16):
                src = v[j]
                for k in range(H // 16):
                    dbuf[vr * 16 + j, pl.ds(k * 16, 16)] = (
                        gbuf[src, pl.ds(k * 16, 16)]
                    )
            return carry

        lax.fori_loop(0, CH // 16, exp_vreg, 0)
        pltpu.sync_copy(dbuf, out_hbm.at[pl.ds(base + cbase, CH)])


def _make_sc_gather(nrows=LEAF, interpret=False):
    return pl.kernel(
        _gather_body,
        out_type=jax.ShapeDtypeStruct((nrows, H), jnp.float32),
        mesh=plsc.VectorSubcoreMesh(
            core_axis_name="c", subcore_axis_name="s",
            num_cores=NC, num_subcores=NS,
        ),
        compiler_params=pltpu.CompilerParams(needs_layout_passes=False),
        scratch_types=[
            pltpu.VMEM((nrows // NW,), jnp.int32),
            pltpu.VMEM((nrows // NW,), jnp.int32),
            pltpu.VMEM((CH,), jnp.int32),
            pltpu.VMEM((CH,), jnp.int32),
            pltpu.VMEM((CH + 8, H), jnp.float32),
            pltpu.VMEM((CH, H), jnp.float32),
            pltpu.SemaphoreType.DMA,
        ],
        interpret=interpret,
    )


def _gates(iou, c_red):
    i = jax.nn.sigmoid(iou[:, :H])
    o = jax.nn.sigmoid(iou[:, H:2 * H])
    u = jnp.tanh(iou[:, 2 * H:])
    c = i * u + c_red
    h = o * jnp.tanh(c)
    return h, c


def _subtree_body(e_ref, m_ref, w_ref, b_ref, wa_ref, ba_ref, lw_ref, lb_ref,
                  *out_refs):
    lw = lw_ref[...]
    lb = lb_ref[...]
    iou = (
        jnp.dot(e_ref[...], w_ref[...], preferred_element_type=jnp.float32)
        * m_ref[...]
        + b_ref[...]
    )
    h, c = _gates(iou, 0.0)
    mh = jnp.maximum(h, 0.0)
    lg0 = jnp.dot(h + mh, lw, preferred_element_type=jnp.float32) + lb
    out_refs[0][...] = lg0.reshape(out_refs[0].shape)
    wa = wa_ref[...]
    ba = ba_ref[...]
    s = SUB
    for k in range(1, D - TOPL):
        s //= 2
        hc = h.reshape(s, 2 * H)
        cc = c.reshape(s, 2 * H)
        mc = mh.reshape(s, 2 * H)
        g = jnp.dot(hc, wa, preferred_element_type=jnp.float32) + ba
        f = jax.nn.sigmoid(g[:, :2 * H])
        c_red = f[:, :H] * cc[:, :H] + f[:, H:] * cc[:, H:]
        h, c = _gates(g[:, 2 * H:], c_red)
        mh = jnp.maximum(h, jnp.maximum(mc[:, :H], mc[:, H:]))
        lg = jnp.dot(h + mh, lw, preferred_element_type=jnp.float32) + lb
        out_refs[k][...] = lg.reshape(out_refs[k].shape)
    out_refs[D - TOPL][...] = h.reshape(1, 1, H)
    out_refs[D - TOPL + 1][...] = c.reshape(1, 1, H)
    out_refs[D - TOPL + 2][...] = mh.reshape(1, 1, H)


@functools.cache
def _make_subtree_call(ncls, nsub=NSUB, interpret=False):
    lg_specs, lg_shapes = [], []
    for l in range(D - 1, TOPL - 1, -1):
        bs = 2 ** (l - TOPL)
        lg_specs.append(pl.BlockSpec((1, bs, ncls), lambda g: (g, 0, 0)))
        lg_shapes.append(jax.ShapeDtypeStruct((nsub, bs, ncls), jnp.float32))
    fr_spec = pl.BlockSpec((1, 1, H), lambda g: (g, 0, 0))
    fr_shape = jax.ShapeDtypeStruct((nsub, 1, H), jnp.float32)
    return pl.pallas_call(
        _subtree_body,
        grid=(nsub,),
        in_specs=[
            pl.BlockSpec((SUB, H), lambda g: (g, 0)),
            pl.BlockSpec((SUB, 1), lambda g: (g, 0)),
            pl.BlockSpec((H, 3 * H), lambda g: (0, 0)),
            pl.BlockSpec((1, 3 * H), lambda g: (0, 0)),
            pl.BlockSpec((2 * H, 5 * H), lambda g: (0, 0)),
            pl.BlockSpec((1, 5 * H), lambda g: (0, 0)),
            pl.BlockSpec((H, ncls), lambda g: (0, 0)),
            pl.BlockSpec((1, ncls), lambda g: (0, 0)),
        ],
        out_specs=[*lg_specs, fr_spec, fr_spec, fr_spec],
        out_shape=[*lg_shapes, fr_shape, fr_shape, fr_shape],
        interpret=interpret,
    )


def _top_body(hc_ref, cc_ref, mc_ref, wa_ref, ba_ref, lw_ref, lb_ref, lg_ref):
    lw = lw_ref[...]
    lb = lb_ref[...]
    wa = wa_ref[...]
    ba = ba_ref[...]
    hc, cc, mc = hc_ref[...], cc_ref[...], mc_ref[...]
    for l in range(TOPL - 1, -1, -1):
        s = 2 ** l
        g = jnp.dot(hc, wa, preferred_element_type=jnp.float32) + ba
        f = jax.nn.sigmoid(g[:, :2 * H])
        c_red = f[:, :H] * cc[:, :H] + f[:, H:] * cc[:, H:]
        h, c = _gates(g[:, 2 * H:], c_red)
        mh = jnp.maximum(h, jnp.maximum(mc[:, :H], mc[:, H:]))
        lg_ref[pl.ds(s - 1, s), :] = (
            jnp.dot(h + mh, lw, preferred_element_type=jnp.float32) + lb
        )
        if l > 0:
            hc = h.reshape(s // 2, 2 * H)
            cc = c.reshape(s // 2, 2 * H)
            mc = mh.reshape(s // 2, 2 * H)


@functools.cache
def _make_top_call(ncls, interpret=False):
    s5 = 2 ** TOPL
    return pl.pallas_call(
        _top_body,
        out_shape=jax.ShapeDtypeStruct((s5 - 1, ncls), jnp.float32),
        interpret=interpret,
    )


def _tree_lstm(leaf_embs, maskfs, w_iou_t, b_iou, w_all, b_all, lin_W, lin_b,
               interpret=False):
    ncls = lin_W.shape[0]
    lw = lin_W.T
    lb = lin_b.reshape(1, ncls)
    nhalf = len(leaf_embs)
    houts = []
    for le, mf in zip(leaf_embs, maskfs):
        nsub = le.shape[0] // SUB
        houts.append(_make_subtree_call(ncls, nsub, interpret)(
            le, mf, w_iou_t, b_iou, w_all, b_all, lw, lb
        ))
    lgs = [jnp.concatenate([o.reshape(-1, ncls) for o in
                            [ho[k] for ho in houts]], axis=0)
           for k in range(D - TOPL)]
    h5 = jnp.concatenate([ho[D - TOPL].reshape(-1, H) for ho in houts], axis=0)
    c5 = jnp.concatenate([ho[D - TOPL + 1].reshape(-1, H) for ho in houts], axis=0)
    mh5 = jnp.concatenate([ho[D - TOPL + 2].reshape(-1, H) for ho in houts], axis=0)
    s5 = 2 ** TOPL
    top_lg = _make_top_call(ncls, interpret)(
        h5.reshape(s5 // 2, 2 * H), c5.reshape(s5 // 2, 2 * H),
        mh5.reshape(s5 // 2, 2 * H), w_all, b_all, lw, lb
    )
    return jnp.concatenate([top_lg, *lgs[::-1]], axis=0)


NHALF = 2


def kernel(wordid, mask, emb, W_iou, U_iou, b_iou, U_f_W, U_f_b, lin_W, lin_b):
    leaf_wid = (wordid[LEAF - 1:] * mask[LEAF - 1:]).astype(jnp.int32)
    maskf = mask[LEAF - 1:].astype(jnp.float32).reshape(LEAF, 1)
    hn = LEAF // NHALF
    msk32 = mask[LEAF - 1:].astype(jnp.int32)
    leaf_embs = [_make_sc_gather(hn)(emb, leaf_wid[i * hn:(i + 1) * hn],
                                     msk32[i * hn:(i + 1) * hn])
                 for i in range(NHALF)]
    maskfs = [maskf[i * hn:(i + 1) * hn] for i in range(NHALF)]
    w_all = jnp.concatenate([U_f_W, U_iou], axis=0).T
    b_all = jnp.concatenate([U_f_b, b_iou[0]]).reshape(1, 5 * H)
    return _tree_lstm(leaf_embs, maskfs, W_iou.T, b_iou, w_all, b_all,
                      lin_W, lin_b)


# speedup vs baseline: 20.5744x; 1.0009x over previous
import functools

import jax
import jax.numpy as jnp
from jax import lax
from jax.experimental import pallas as pl
from jax.experimental.pallas import tpu as pltpu
from jax.experimental.pallas import tpu_sc as plsc

H = 128
D = 17
N = 2**D - 1
LEAF = 2 ** (D - 1)

NC, NS = 2, 16
NW = NC * NS
ROWS_W = LEAF // NW
BIG = 512

SUB = 2048
NSUB = LEAF // SUB
TOPL = 5


CH = 256
GU = 64
SENT = CH


def _gather_body(emb_hbm, idx_hbm, msk_hbm, out_hbm,
                 idx_v, msk_v, cidx_v, smap_v, gbuf, dbuf, sem):
    rows_w = idx_hbm.shape[0] // NW
    wid = lax.axis_index("s") * NC + lax.axis_index("c")
    base = wid * rows_w
    pltpu.sync_copy(idx_hbm.at[pl.ds(base, rows_w)], idx_v)
    pltpu.sync_copy(msk_hbm.at[pl.ds(base, rows_w)], msk_v)
    for v in range(CH // 16):
        cidx_v[pl.ds(v * 16, 16)] = jnp.zeros((16,), jnp.int32)
    for k in range(H // 16):
        gbuf[SENT, pl.ds(k * 16, 16)] = jnp.zeros((16,), jnp.float32)
    for c in range(rows_w // CH):
        cbase = c * CH
        cnt = jnp.int32(0)
        for v in range(CH // 16):
            m = msk_v[pl.ds(cbase + v * 16, 16)]
            w = idx_v[pl.ds(cbase + v * 16, 16)]
            mb = m > 0
            pos = plsc.cumsum(m) + (cnt - 1)
            smap_v[pl.ds(v * 16, 16)] = jnp.where(mb, pos, SENT)
            plsc.store_scatter(cidx_v, [pos], w, mask=mb)
            cnt = cnt + jnp.sum(m)

        n_u = (cnt + (GU - 1)) // GU

        def gath_fire(i, carry):
            pltpu.async_copy(
                emb_hbm.at[cidx_v.at[pl.ds(i * GU, GU)]],
                gbuf.at[pl.ds(i * GU, GU)], sem,
            )
            return carry

        def gath_drain(i, carry):
            pltpu.make_async_copy(
                emb_hbm.at[cidx_v.at[pl.ds(i * GU, GU)]],
                gbuf.at[pl.ds(i * GU, GU)], sem,
            ).wait()
            return carry

        lax.fori_loop(0, n_u, gath_fire, 0)
        lax.fori_loop(0, n_u, gath_drain, 0)

        def exp_vreg(vr, carry):
            v = smap_v[pl.ds(vr * 16, 16)]
            for j in range(16):
                src = v[j]
                for k in range(H // 16):
                    dbuf[vr * 16 + j, pl.ds(k * 16, 16)] = (
                        gbuf[src, pl.ds(k * 16, 16)]
                    )
            return carry

        lax.fori_loop(0, CH // 16, exp_vreg, 0)
        pltpu.sync_copy(dbuf, out_hbm.at[pl.ds(base + cbase, CH)])


def _make_sc_gather(nrows=LEAF, interpret=False):
    return pl.kernel(
        _gather_body,
        out_type=jax.ShapeDtypeStruct((nrows, H), jnp.float32),
        mesh=plsc.VectorSubcoreMesh(
            core_axis_name="c", subcore_axis_name="s",
            num_cores=NC, num_subcores=NS,
        ),
        compiler_params=pltpu.CompilerParams(needs_layout_passes=False),
        scratch_types=[
            pltpu.VMEM((nrows // NW,), jnp.int32),
            pltpu.VMEM((nrows // NW,), jnp.int32),
            pltpu.VMEM((CH,), jnp.int32),
            pltpu.VMEM((CH,), jnp.int32),
            pltpu.VMEM((CH + 8, H), jnp.float32),
            pltpu.VMEM((CH, H), jnp.float32),
            pltpu.SemaphoreType.DMA,
        ],
        interpret=interpret,
    )


def _gates(iou, c_red):
    i = jax.nn.sigmoid(iou[:, :H])
    o = jax.nn.sigmoid(iou[:, H:2 * H])
    u = jnp.tanh(iou[:, 2 * H:])
    c = i * u + c_red
    h = o * jnp.tanh(c)
    return h, c


def _subtree_body(e_ref, m_ref, w_ref, b_ref, wa_ref, ba_ref, lw_ref, lb_ref,
                  *out_refs):
    lw = lw_ref[...]
    lb = lb_ref[...]
    iou = (
        jnp.dot(e_ref[...], w_ref[...], preferred_element_type=jnp.float32)
        * m_ref[...]
        + b_ref[...]
    )
    h, c = _gates(iou, 0.0)
    mh = jnp.maximum(h, 0.0)
    lg0 = jnp.dot(h + mh, lw, preferred_element_type=jnp.float32) + lb
    out_refs[0][...] = lg0.reshape(out_refs[0].shape)
    wa = wa_ref[...]
    ba = ba_ref[...]
    s = SUB
    for k in range(1, D - TOPL):
        s //= 2
        hc = h.reshape(s, 2 * H)
        cc = c.reshape(s, 2 * H)
        mc = mh.reshape(s, 2 * H)
        g = jnp.dot(hc, wa, preferred_element_type=jnp.float32) + ba
        f = jax.nn.sigmoid(g[:, :2 * H])
        c_red = f[:, :H] * cc[:, :H] + f[:, H:] * cc[:, H:]
        h, c = _gates(g[:, 2 * H:], c_red)
        mh = jnp.maximum(h, jnp.maximum(mc[:, :H], mc[:, H:]))
        lg = jnp.dot(h + mh, lw, preferred_element_type=jnp.float32) + lb
        out_refs[k][...] = lg.reshape(out_refs[k].shape)
    out_refs[D - TOPL][...] = h.reshape(1, 1, H)
    out_refs[D - TOPL + 1][...] = c.reshape(1, 1, H)
    out_refs[D - TOPL + 2][...] = mh.reshape(1, 1, H)


@functools.cache
def _make_subtree_call(ncls, nsub=NSUB, interpret=False):
    lg_specs, lg_shapes = [], []
    for l in range(D - 1, TOPL - 1, -1):
        bs = 2 ** (l - TOPL)
        lg_specs.append(pl.BlockSpec((1, bs, ncls), lambda g: (g, 0, 0)))
        lg_shapes.append(jax.ShapeDtypeStruct((nsub, bs, ncls), jnp.float32))
    fr_spec = pl.BlockSpec((1, 1, H), lambda g: (g, 0, 0))
    fr_shape = jax.ShapeDtypeStruct((nsub, 1, H), jnp.float32)
    return pl.pallas_call(
        _subtree_body,
        grid=(nsub,),
        in_specs=[
            pl.BlockSpec((SUB, H), lambda g: (g, 0)),
            pl.BlockSpec((SUB, 1), lambda g: (g, 0)),
            pl.BlockSpec((H, 3 * H), lambda g: (0, 0)),
            pl.BlockSpec((1, 3 * H), lambda g: (0, 0)),
            pl.BlockSpec((2 * H, 5 * H), lambda g: (0, 0)),
            pl.BlockSpec((1, 5 * H), lambda g: (0, 0)),
            pl.BlockSpec((H, ncls), lambda g: (0, 0)),
            pl.BlockSpec((1, ncls), lambda g: (0, 0)),
        ],
        out_specs=[*lg_specs, fr_spec, fr_spec, fr_spec],
        out_shape=[*lg_shapes, fr_shape, fr_shape, fr_shape],
        interpret=interpret,
    )


def _top_body(hc_ref, cc_ref, mc_ref, wa_ref, ba_ref, lw_ref, lb_ref, lg_ref):
    lw = lw_ref[...]
    lb = lb_ref[...]
    wa = wa_ref[...]
    ba = ba_ref[...]
    hc, cc, mc = hc_ref[...], cc_ref[...], mc_ref[...]
    for l in range(TOPL - 1, -1, -1):
        s = 2 ** l
        g = jnp.dot(hc, wa, preferred_element_type=jnp.float32) + ba
        f = jax.nn.sigmoid(g[:, :2 * H])
        c_red = f[:, :H] * cc[:, :H] + f[:, H:] * cc[:, H:]
        h, c = _gates(g[:, 2 * H:], c_red)
        mh = jnp.maximum(h, jnp.maximum(mc[:, :H], mc[:, H:]))
        lg_ref[pl.ds(s - 1, s), :] = (
            jnp.dot(h + mh, lw, preferred_element_type=jnp.float32) + lb
        )
        if l > 0:
            hc = h.reshape(s // 2, 2 * H)
            cc = c.reshape(s // 2, 2 * H)
            mc = mh.reshape(s // 2, 2 * H)


@functools.cache
def _make_top_call(ncls, interpret=False):
    s5 = 2 ** TOPL
    return pl.pallas_call(
        _top_body,
        out_shape=jax.ShapeDtypeStruct((s5 - 1, ncls), jnp.float32),
        interpret=interpret,
    )


def _tree_lstm(leaf_embs, maskfs, w_iou_t, b_iou, w_all, b_all, lin_W, lin_b,
               interpret=False):
    ncls = lin_W.shape[0]
    lw = lin_W.T
    lb = lin_b.reshape(1, ncls)
    nhalf = len(leaf_embs)
    houts = []
    for le, mf in zip(leaf_embs, maskfs):
        nsub = le.shape[0] // SUB
        houts.append(_make_subtree_call(ncls, nsub, interpret)(
            le, mf, w_iou_t, b_iou, w_all, b_all, lw, lb
        ))
    lgs = [jnp.concatenate([o.reshape(-1, ncls) for o in
                            [ho[k] for ho in houts]], axis=0)
           for k in range(D - TOPL)]
    h5 = jnp.concatenate([ho[D - TOPL].reshape(-1, H) for ho in houts], axis=0)
    c5 = jnp.concatenate([ho[D - TOPL + 1].reshape(-1, H) for ho in houts], axis=0)
    mh5 = jnp.concatenate([ho[D - TOPL + 2].reshape(-1, H) for ho in houts], axis=0)
    s5 = 2 ** TOPL
    top_lg = _make_top_call(ncls, interpret)(
        h5.reshape(s5 // 2, 2 * H), c5.reshape(s5 // 2, 2 * H),
        mh5.reshape(s5 // 2, 2 * H), w_all, b_all, lw, lb
    )
    return jnp.concatenate([top_lg, *lgs[::-1]], axis=0)


NHALF = 2


def kernel(wordid, mask, emb, W_iou, U_iou, b_iou, U_f_W, U_f_b, lin_W, lin_b):
    leaf_wid = (wordid[LEAF - 1:] * mask[LEAF - 1:]).astype(jnp.int32)
    maskf = mask[LEAF - 1:].astype(jnp.float32).reshape(LEAF, 1)
    hn = LEAF // NHALF
    msk32 = mask[LEAF - 1:].astype(jnp.int32)
    leaf_embs = [_make_sc_gather(hn)(emb, leaf_wid[i * hn:(i + 1) * hn],
                                     msk32[i * hn:(i + 1) * hn])
                 for i in range(NHALF)]
    maskfs = [maskf[i * hn:(i + 1) * hn] for i in range(NHALF)]
    w_all = jnp.concatenate([U_f_W, U_iou], axis=0).T
    b_all = jnp.concatenate([U_f_b, b_iou[0]]).reshape(1, 5 * H)
    return _tree_lstm(leaf_embs, maskfs, W_iou.T, b_iou, w_all, b_all,
                      lin_W, lin_b)


# speedup vs baseline: 27.6749x; 1.3451x over previous
import functools

import jax
import jax.numpy as jnp
from jax import lax
from jax.experimental import pallas as pl
from jax.experimental.pallas import tpu as pltpu
from jax.experimental.pallas import tpu_sc as plsc

H = 128
D = 17
N = 2**D - 1
LEAF = 2 ** (D - 1)

NC, NS = 2, 16
NW = NC * NS
ROWS_W = LEAF // NW
BIG = 512

SUB = 2048
NSUB = LEAF // SUB
TOPL = 5


CH = 256
GU = 32
SENT = CH


def _gather_body(emb_hbm, idx_hbm, msk_hbm, out_hbm,
                 idx_v, msk_v, cidx_v, smap_v, gbuf, dbuf, sem):
    rows_w = idx_hbm.shape[0] // NW
    wid = lax.axis_index("s") * NC + lax.axis_index("c")
    base = wid * rows_w
    pltpu.sync_copy(idx_hbm.at[pl.ds(base, rows_w)], idx_v)
    pltpu.sync_copy(msk_hbm.at[pl.ds(base, rows_w)], msk_v)
    for v in range(CH // 16):
        cidx_v[pl.ds(v * 16, 16)] = jnp.zeros((16,), jnp.int32)
    for k in range(H // 16):
        gbuf[SENT, pl.ds(k * 16, 16)] = jnp.zeros((16,), jnp.float32)
    for c in range(rows_w // CH):
        cbase = c * CH
        cnt = jnp.int32(0)
        for v in range(CH // 16):
            m = msk_v[pl.ds(cbase + v * 16, 16)]
            w = idx_v[pl.ds(cbase + v * 16, 16)]
            mb = m > 0
            pos = plsc.cumsum(m) + (cnt - 1)
            smap_v[pl.ds(v * 16, 16)] = jnp.where(mb, pos, SENT)
            plsc.store_scatter(cidx_v, [pos], w, mask=mb)
            cnt = cnt + jnp.sum(m)

        n_u = (cnt + (GU - 1)) // GU

        def gath_fire(i, carry):
            pltpu.async_copy(
                emb_hbm.at[cidx_v.at[pl.ds(i * GU, GU)]],
                gbuf.at[pl.ds(i * GU, GU)], sem,
            )
            return carry

        def gath_drain(i, carry):
            pltpu.make_async_copy(
                emb_hbm.at[cidx_v.at[pl.ds(i * GU, GU)]],
                gbuf.at[pl.ds(i * GU, GU)], sem,
            ).wait()
            return carry

        lax.fori_loop(0, n_u, gath_fire, 0)
        lax.fori_loop(0, n_u, gath_drain, 0)

        def exp_vreg(vr, carry):
            v = smap_v[pl.ds(vr * 16, 16)]
            for j in range(16):
                src = v[j]
                for k in range(H // 16):
                    dbuf[vr * 16 + j, pl.ds(k * 16, 16)] = (
                        gbuf[src, pl.ds(k * 16, 16)]
                    )
            return carry

        lax.fori_loop(0, CH // 16, exp_vreg, 0)
        pltpu.sync_copy(dbuf, out_hbm.at[pl.ds(base + cbase, CH)])


def _make_sc_gather(nrows=LEAF, interpret=False):
    return pl.kernel(
        _gather_body,
        out_type=jax.ShapeDtypeStruct((nrows, H), jnp.float32),
        mesh=plsc.VectorSubcoreMesh(
            core_axis_name="c", subcore_axis_name="s",
            num_cores=NC, num_subcores=NS,
        ),
        compiler_params=pltpu.CompilerParams(needs_layout_passes=False),
        scratch_types=[
            pltpu.VMEM((nrows // NW,), jnp.int32),
            pltpu.VMEM((nrows // NW,), jnp.int32),
            pltpu.VMEM((CH,), jnp.int32),
            pltpu.VMEM((CH,), jnp.int32),
            pltpu.VMEM((CH + 8, H), jnp.float32),
            pltpu.VMEM((CH, H), jnp.float32),
            pltpu.SemaphoreType.DMA,
        ],
        interpret=interpret,
    )


def _gates(iou, c_red):
    i = jax.nn.sigmoid(iou[:, :H])
    o = jax.nn.sigmoid(iou[:, H:2 * H])
    u = jnp.tanh(iou[:, 2 * H:])
    c = i * u + c_red
    h = o * jnp.tanh(c)
    return h, c


def _subtree_body(e_ref, m_ref, w_ref, b_ref, wa_ref, ba_ref, lw_ref, lb_ref,
                  *out_refs):
    lw = lw_ref[...]
    lb = lb_ref[...]
    iou = (
        jnp.dot(e_ref[...], w_ref[...], preferred_element_type=jnp.float32)
        * m_ref[...]
        + b_ref[...]
    )
    h, c = _gates(iou, 0.0)
    mh = jnp.maximum(h, 0.0)
    lg0 = jnp.dot(h + mh, lw, preferred_element_type=jnp.float32) + lb
    out_refs[0][...] = lg0.reshape(out_refs[0].shape)
    wa = wa_ref[...]
    ba = ba_ref[...]
    s = SUB
    for k in range(1, D - TOPL):
        s //= 2
        hc = h.reshape(s, 2 * H)
        cc = c.reshape(s, 2 * H)
        mc = mh.reshape(s, 2 * H)
        g = jnp.dot(hc, wa, preferred_element_type=jnp.float32) + ba
        f = jax.nn.sigmoid(g[:, :2 * H])
        c_red = f[:, :H] * cc[:, :H] + f[:, H:] * cc[:, H:]
        h, c = _gates(g[:, 2 * H:], c_red)
        mh = jnp.maximum(h, jnp.maximum(mc[:, :H], mc[:, H:]))
        lg = jnp.dot(h + mh, lw, preferred_element_type=jnp.float32) + lb
        out_refs[k][...] = lg.reshape(out_refs[k].shape)
    out_refs[D - TOPL][...] = h.reshape(1, 1, H)
    out_refs[D - TOPL + 1][...] = c.reshape(1, 1, H)
    out_refs[D - TOPL + 2][...] = mh.reshape(1, 1, H)


@functools.cache
def _make_subtree_call(ncls, nsub=NSUB, interpret=False):
    lg_specs, lg_shapes = [], []
    for l in range(D - 1, TOPL - 1, -1):
        bs = 2 ** (l - TOPL)
        lg_specs.append(pl.BlockSpec((1, bs, ncls), lambda g: (g, 0, 0)))
        lg_shapes.append(jax.ShapeDtypeStruct((nsub, bs, ncls), jnp.float32))
    fr_spec = pl.BlockSpec((1, 1, H), lambda g: (g, 0, 0))
    fr_shape = jax.ShapeDtypeStruct((nsub, 1, H), jnp.float32)
    return pl.pallas_call(
        _subtree_body,
        grid=(nsub,),
        in_specs=[
            pl.BlockSpec((SUB, H), lambda g: (g, 0)),
            pl.BlockSpec((SUB, 1), lambda g: (g, 0)),
            pl.BlockSpec((H, 3 * H), lambda g: (0, 0)),
            pl.BlockSpec((1, 3 * H), lambda g: (0, 0)),
            pl.BlockSpec((2 * H, 5 * H), lambda g: (0, 0)),
            pl.BlockSpec((1, 5 * H), lambda g: (0, 0)),
            pl.BlockSpec((H, ncls), lambda g: (0, 0)),
            pl.BlockSpec((1, ncls), lambda g: (0, 0)),
        ],
        out_specs=[*lg_specs, fr_spec, fr_spec, fr_spec],
        out_shape=[*lg_shapes, fr_shape, fr_shape, fr_shape],
        interpret=interpret,
    )


def _top_body(hc_ref, cc_ref, mc_ref, wa_ref, ba_ref, lw_ref, lb_ref, lg_ref):
    lw = lw_ref[...]
    lb = lb_ref[...]
    wa = wa_ref[...]
    ba = ba_ref[...]
    hc, cc, mc = hc_ref[...], cc_ref[...], mc_ref[...]
    for l in range(TOPL - 1, -1, -1):
        s = 2 ** l
        g = jnp.dot(hc, wa, preferred_element_type=jnp.float32) + ba
        f = jax.nn.sigmoid(g[:, :2 * H])
        c_red = f[:, :H] * cc[:, :H] + f[:, H:] * cc[:, H:]
        h, c = _gates(g[:, 2 * H:], c_red)
        mh = jnp.maximum(h, jnp.maximum(mc[:, :H], mc[:, H:]))
        lg_ref[pl.ds(s - 1, s), :] = (
            jnp.dot(h + mh, lw, preferred_element_type=jnp.float32) + lb
        )
        if l > 0:
            hc = h.reshape(s // 2, 2 * H)
            cc = c.reshape(s // 2, 2 * H)
            mc = mh.reshape(s // 2, 2 * H)


@functools.cache
def _make_top_call(ncls, interpret=False):
    s5 = 2 ** TOPL
    return pl.pallas_call(
        _top_body,
        out_shape=jax.ShapeDtypeStruct((s5 - 1, ncls), jnp.float32),
        interpret=interpret,
    )


def _tree_lstm(leaf_embs, maskfs, w_iou_t, b_iou, w_all, b_all, lin_W, lin_b,
               interpret=False):
    ncls = lin_W.shape[0]
    lw = lin_W.T
    lb = lin_b.reshape(1, ncls)
    nhalf = len(leaf_embs)
    houts = []
    for le, mf in zip(leaf_embs, maskfs):
        nsub = le.shape[0] // SUB
        houts.append(_make_subtree_call(ncls, nsub, interpret)(
            le, mf, w_iou_t, b_iou, w_all, b_all, lw, lb
        ))
    lgs = [jnp.concatenate([o.reshape(-1, ncls) for o in
                            [ho[k] for ho in houts]], axis=0)
           for k in range(D - TOPL)]
    h5 = jnp.concatenate([ho[D - TOPL].reshape(-1, H) for ho in houts], axis=0)
    c5 = jnp.concatenate([ho[D - TOPL + 1].reshape(-1, H) for ho in houts], axis=0)
    mh5 = jnp.concatenate([ho[D - TOPL + 2].reshape(-1, H) for ho in houts], axis=0)
    s5 = 2 ** TOPL
    top_lg = _make_top_call(ncls, interpret)(
        h5.reshape(s5 // 2, 2 * H), c5.reshape(s5 // 2, 2 * H),
        mh5.reshape(s5 // 2, 2 * H), w_all, b_all, lw, lb
    )
    return jnp.concatenate([top_lg, *lgs[::-1]], axis=0)


NHALF = 2


def kernel(wordid, mask, emb, W_iou, U_iou, b_iou, U_f_W, U_f_b, lin_W, lin_b):
    leaf_wid = (wordid[LEAF - 1:] * mask[LEAF - 1:]).astype(jnp.int32)
    maskf = mask[LEAF - 1:].astype(jnp.float32).reshape(LEAF, 1)
    hn = LEAF // NHALF
    msk32 = mask[LEAF - 1:].astype(jnp.int32)
    leaf_embs = [_make_sc_gather(hn)(emb, leaf_wid[i * hn:(i + 1) * hn],
                                     msk32[i * hn:(i + 1) * hn])
                 for i in range(NHALF)]
    maskfs = [maskf[i * hn:(i + 1) * hn] for i in range(NHALF)]
    w_all = jnp.concatenate([U_f_W, U_iou], axis=0).T
    b_all = jnp.concatenate([U_f_b, b_iou[0]]).reshape(1, 5 * H)
    return _tree_lstm(leaf_embs, maskfs, W_iou.T, b_iou, w_all, b_all,
                      lin_W, lin_b)


# speedup vs baseline: 32.0564x; 1.1583x over previous
import functools

import jax
import jax.numpy as jnp
from jax import lax
from jax.experimental import pallas as pl
from jax.experimental.pallas import tpu as pltpu
from jax.experimental.pallas import tpu_sc as plsc

H = 128
D = 17
N = 2**D - 1
LEAF = 2 ** (D - 1)

NC, NS = 2, 16
NW = NC * NS
ROWS_W = LEAF // NW
BIG = 512

SUB = 2048
NSUB = LEAF // SUB
TOPL = 5


CH = 256
GU = 16
SENT = CH


def _gather_body(emb_hbm, idx_hbm, msk_hbm, out_hbm,
                 idx_v, msk_v, cidx_v, smap_v, gbuf, dbuf, sem):
    rows_w = idx_hbm.shape[0] // NW
    wid = lax.axis_index("s") * NC + lax.axis_index("c")
    base = wid * rows_w
    pltpu.sync_copy(idx_hbm.at[pl.ds(base, rows_w)], idx_v)
    pltpu.sync_copy(msk_hbm.at[pl.ds(base, rows_w)], msk_v)
    for v in range(CH // 16):
        cidx_v[pl.ds(v * 16, 16)] = jnp.zeros((16,), jnp.int32)
    for k in range(H // 16):
        gbuf[SENT, pl.ds(k * 16, 16)] = jnp.zeros((16,), jnp.float32)
    for c in range(rows_w // CH):
        cbase = c * CH
        cnt = jnp.int32(0)
        for v in range(CH // 16):
            m = msk_v[pl.ds(cbase + v * 16, 16)]
            w = idx_v[pl.ds(cbase + v * 16, 16)]
            mb = m > 0
            pos = plsc.cumsum(m) + (cnt - 1)
            smap_v[pl.ds(v * 16, 16)] = jnp.where(mb, pos, SENT)
            plsc.store_scatter(cidx_v, [pos], w, mask=mb)
            cnt = cnt + jnp.sum(m)

        n_u = (cnt + (GU - 1)) // GU

        def gath_fire(i, carry):
            pltpu.async_copy(
                emb_hbm.at[cidx_v.at[pl.ds(i * GU, GU)]],
                gbuf.at[pl.ds(i * GU, GU)], sem,
            )
            return carry

        def gath_drain(i, carry):
            pltpu.make_async_copy(
                emb_hbm.at[cidx_v.at[pl.ds(i * GU, GU)]],
                gbuf.at[pl.ds(i * GU, GU)], sem,
            ).wait()
            return carry

        lax.fori_loop(0, n_u, gath_fire, 0)
        lax.fori_loop(0, n_u, gath_drain, 0)

        def exp_vreg(vr, carry):
            v = smap_v[pl.ds(vr * 16, 16)]
            for j in range(16):
                src = v[j]
                for k in range(H // 16):
                    dbuf[vr * 16 + j, pl.ds(k * 16, 16)] = (
                        gbuf[src, pl.ds(k * 16, 16)]
                    )
            return carry

        lax.fori_loop(0, CH // 16, exp_vreg, 0)
        pltpu.sync_copy(dbuf, out_hbm.at[pl.ds(base + cbase, CH)])


def _make_sc_gather(nrows=LEAF, interpret=False):
    return pl.kernel(
        _gather_body,
        out_type=jax.ShapeDtypeStruct((nrows, H), jnp.float32),
        mesh=plsc.VectorSubcoreMesh(
            core_axis_name="c", subcore_axis_name="s",
            num_cores=NC, num_subcores=NS,
        ),
        compiler_params=pltpu.CompilerParams(needs_layout_passes=False),
        scratch_types=[
            pltpu.VMEM((nrows // NW,), jnp.int32),
            pltpu.VMEM((nrows // NW,), jnp.int32),
            pltpu.VMEM((CH,), jnp.int32),
            pltpu.VMEM((CH,), jnp.int32),
            pltpu.VMEM((CH + 8, H), jnp.float32),
            pltpu.VMEM((CH, H), jnp.float32),
            pltpu.SemaphoreType.DMA,
        ],
        interpret=interpret,
    )


def _gates(iou, c_red):
    i = jax.nn.sigmoid(iou[:, :H])
    o = jax.nn.sigmoid(iou[:, H:2 * H])
    u = jnp.tanh(iou[:, 2 * H:])
    c = i * u + c_red
    h = o * jnp.tanh(c)
    return h, c


def _subtree_body(e_ref, m_ref, w_ref, b_ref, wa_ref, ba_ref, lw_ref, lb_ref,
                  *out_refs):
    lw = lw_ref[...]
    lb = lb_ref[...]
    iou = (
        jnp.dot(e_ref[...], w_ref[...], preferred_element_type=jnp.float32)
        * m_ref[...]
        + b_ref[...]
    )
    h, c = _gates(iou, 0.0)
    mh = jnp.maximum(h, 0.0)
    lg0 = jnp.dot(h + mh, lw, preferred_element_type=jnp.float32) + lb
    out_refs[0][...] = lg0.reshape(out_refs[0].shape)
    wa = wa_ref[...]
    ba = ba_ref[...]
    s = SUB
    for k in range(1, D - TOPL):
        s //= 2
        hc = h.reshape(s, 2 * H)
        cc = c.reshape(s, 2 * H)
        mc = mh.reshape(s, 2 * H)
        g = jnp.dot(hc, wa, preferred_element_type=jnp.float32) + ba
        f = jax.nn.sigmoid(g[:, :2 * H])
        c_red = f[:, :H] * cc[:, :H] + f[:, H:] * cc[:, H:]
        h, c = _gates(g[:, 2 * H:], c_red)
        mh = jnp.maximum(h, jnp.maximum(mc[:, :H], mc[:, H:]))
        lg = jnp.dot(h + mh, lw, preferred_element_type=jnp.float32) + lb
        out_refs[k][...] = lg.reshape(out_refs[k].shape)
    out_refs[D - TOPL][...] = h.reshape(1, 1, H)
    out_refs[D - TOPL + 1][...] = c.reshape(1, 1, H)
    out_refs[D - TOPL + 2][...] = mh.reshape(1, 1, H)


@functools.cache
def _make_subtree_call(ncls, nsub=NSUB, interpret=False):
    lg_specs, lg_shapes = [], []
    for l in range(D - 1, TOPL - 1, -1):
        bs = 2 ** (l - TOPL)
        lg_specs.append(pl.BlockSpec((1, bs, ncls), lambda g: (g, 0, 0)))
        lg_shapes.append(jax.ShapeDtypeStruct((nsub, bs, ncls), jnp.float32))
    fr_spec = pl.BlockSpec((1, 1, H), lambda g: (g, 0, 0))
    fr_shape = jax.ShapeDtypeStruct((nsub, 1, H), jnp.float32)
    return pl.pallas_call(
        _subtree_body,
        grid=(nsub,),
        in_specs=[
            pl.BlockSpec((SUB, H), lambda g: (g, 0)),
            pl.BlockSpec((SUB, 1), lambda g: (g, 0)),
            pl.BlockSpec((H, 3 * H), lambda g: (0, 0)),
            pl.BlockSpec((1, 3 * H), lambda g: (0, 0)),
            pl.BlockSpec((2 * H, 5 * H), lambda g: (0, 0)),
            pl.BlockSpec((1, 5 * H), lambda g: (0, 0)),
            pl.BlockSpec((H, ncls), lambda g: (0, 0)),
            pl.BlockSpec((1, ncls), lambda g: (0, 0)),
        ],
        out_specs=[*lg_specs, fr_spec, fr_spec, fr_spec],
        out_shape=[*lg_shapes, fr_shape, fr_shape, fr_shape],
        interpret=interpret,
    )


def _top_body(hc_ref, cc_ref, mc_ref, wa_ref, ba_ref, lw_ref, lb_ref, lg_ref):
    lw = lw_ref[...]
    lb = lb_ref[...]
    wa = wa_ref[...]
    ba = ba_ref[...]
    hc, cc, mc = hc_ref[...], cc_ref[...], mc_ref[...]
    for l in range(TOPL - 1, -1, -1):
        s = 2 ** l
        g = jnp.dot(hc, wa, preferred_element_type=jnp.float32) + ba
        f = jax.nn.sigmoid(g[:, :2 * H])
        c_red = f[:, :H] * cc[:, :H] + f[:, H:] * cc[:, H:]
        h, c = _gates(g[:, 2 * H:], c_red)
        mh = jnp.maximum(h, jnp.maximum(mc[:, :H], mc[:, H:]))
        lg_ref[pl.ds(s - 1, s), :] = (
            jnp.dot(h + mh, lw, preferred_element_type=jnp.float32) + lb
        )
        if l > 0:
            hc = h.reshape(s // 2, 2 * H)
            cc = c.reshape(s // 2, 2 * H)
            mc = mh.reshape(s // 2, 2 * H)


@functools.cache
def _make_top_call(ncls, interpret=False):
    s5 = 2 ** TOPL
    return pl.pallas_call(
        _top_body,
        out_shape=jax.ShapeDtypeStruct((s5 - 1, ncls), jnp.float32),
        interpret=interpret,
    )


def _tree_lstm(leaf_embs, maskfs, w_iou_t, b_iou, w_all, b_all, lin_W, lin_b,
               interpret=False):
    ncls = lin_W.shape[0]
    lw = lin_W.T
    lb = lin_b.reshape(1, ncls)
    nhalf = len(leaf_embs)
    houts = []
    for le, mf in zip(leaf_embs, maskfs):
        nsub = le.shape[0] // SUB
        houts.append(_make_subtree_call(ncls, nsub, interpret)(
            le, mf, w_iou_t, b_iou, w_all, b_all, lw, lb
        ))
    lgs = [jnp.concatenate([o.reshape(-1, ncls) for o in
                            [ho[k] for ho in houts]], axis=0)
           for k in range(D - TOPL)]
    h5 = jnp.concatenate([ho[D - TOPL].reshape(-1, H) for ho in houts], axis=0)
    c5 = jnp.concatenate([ho[D - TOPL + 1].reshape(-1, H) for ho in houts], axis=0)
    mh5 = jnp.concatenate([ho[D - TOPL + 2].reshape(-1, H) for ho in houts], axis=0)
    s5 = 2 ** TOPL
    top_lg = _make_top_call(ncls, interpret)(
        h5.reshape(s5 // 2, 2 * H), c5.reshape(s5 // 2, 2 * H),
        mh5.reshape(s5 // 2, 2 * H), w_all, b_all, lw, lb
    )
    return jnp.concatenate([top_lg, *lgs[::-1]], axis=0)


NHALF = 2


def kernel(wordid, mask, emb, W_iou, U_iou, b_iou, U_f_W, U_f_b, lin_W, lin_b):
    leaf_wid = (wordid[LEAF - 1:] * mask[LEAF - 1:]).astype(jnp.int32)
    maskf = mask[LEAF - 1:].astype(jnp.float32).reshape(LEAF, 1)
    hn = LEAF // NHALF
    msk32 = mask[LEAF - 1:].astype(jnp.int32)
    leaf_embs = [_make_sc_gather(hn)(emb, leaf_wid[i * hn:(i + 1) * hn],
                                     msk32[i * hn:(i + 1) * hn])
                 for i in range(NHALF)]
    maskfs = [maskf[i * hn:(i + 1) * hn] for i in range(NHALF)]
    w_all = jnp.concatenate([U_f_W, U_iou], axis=0).T
    b_all = jnp.concatenate([U_f_b, b_iou[0]]).reshape(1, 5 * H)
    return _tree_lstm(leaf_embs, maskfs, W_iou.T, b_iou, w_all, b_all,
                      lin_W, lin_b)


# speedup vs baseline: 34.0514x; 1.0622x over previous
import functools

import jax
import jax.numpy as jnp
from jax import lax
from jax.experimental import pallas as pl
from jax.experimental.pallas import tpu as pltpu
from jax.experimental.pallas import tpu_sc as plsc

H = 128
D = 17
N = 2**D - 1
LEAF = 2 ** (D - 1)

NC, NS = 2, 16
NW = NC * NS
ROWS_W = LEAF // NW
BIG = 512

SUB = 2048
NSUB = LEAF // SUB
TOPL = 5


CH = 256
GU = 8
SENT = CH


def _gather_body(emb_hbm, idx_hbm, msk_hbm, out_hbm,
                 idx_v, msk_v, cidx_v, smap_v, gbuf, dbuf, sem):
    rows_w = idx_hbm.shape[0] // NW
    wid = lax.axis_index("s") * NC + lax.axis_index("c")
    base = wid * rows_w
    pltpu.sync_copy(idx_hbm.at[pl.ds(base, rows_w)], idx_v)
    pltpu.sync_copy(msk_hbm.at[pl.ds(base, rows_w)], msk_v)
    for v in range(CH // 16):
        cidx_v[pl.ds(v * 16, 16)] = jnp.zeros((16,), jnp.int32)
    for k in range(H // 16):
        gbuf[SENT, pl.ds(k * 16, 16)] = jnp.zeros((16,), jnp.float32)
    for c in range(rows_w // CH):
        cbase = c * CH
        cnt = jnp.int32(0)
        for v in range(CH // 16):
            m = msk_v[pl.ds(cbase + v * 16, 16)]
            w = idx_v[pl.ds(cbase + v * 16, 16)]
            mb = m > 0
            pos = plsc.cumsum(m) + (cnt - 1)
            smap_v[pl.ds(v * 16, 16)] = jnp.where(mb, pos, SENT)
            plsc.store_scatter(cidx_v, [pos], w, mask=mb)
            cnt = cnt + jnp.sum(m)

        n_u = (cnt + (GU - 1)) // GU

        def gath_fire(i, carry):
            pltpu.async_copy(
                emb_hbm.at[cidx_v.at[pl.ds(i * GU, GU)]],
                gbuf.at[pl.ds(i * GU, GU)], sem,
            )
            return carry

        def gath_drain(i, carry):
            pltpu.make_async_copy(
                emb_hbm.at[cidx_v.at[pl.ds(i * GU, GU)]],
                gbuf.at[pl.ds(i * GU, GU)], sem,
            ).wait()
            return carry

        lax.fori_loop(0, n_u, gath_fire, 0)
        lax.fori_loop(0, n_u, gath_drain, 0)

        def exp_vreg(vr, carry):
            v = smap_v[pl.ds(vr * 16, 16)]
            for j in range(16):
                src = v[j]
                for k in range(H // 16):
                    dbuf[vr * 16 + j, pl.ds(k * 16, 16)] = (
                        gbuf[src, pl.ds(k * 16, 16)]
                    )
            return carry

        lax.fori_loop(0, CH // 16, exp_vreg, 0)
        pltpu.sync_copy(dbuf, out_hbm.at[pl.ds(base + cbase, CH)])


def _make_sc_gather(nrows=LEAF, interpret=False):
    return pl.kernel(
        _gather_body,
        out_type=jax.ShapeDtypeStruct((nrows, H), jnp.float32),
        mesh=plsc.VectorSubcoreMesh(
            core_axis_name="c", subcore_axis_name="s",
            num_cores=NC, num_subcores=NS,
        ),
        compiler_params=pltpu.CompilerParams(needs_layout_passes=False),
        scratch_types=[
            pltpu.VMEM((nrows // NW,), jnp.int32),
            pltpu.VMEM((nrows // NW,), jnp.int32),
            pltpu.VMEM((CH,), jnp.int32),
            pltpu.VMEM((CH,), jnp.int32),
            pltpu.VMEM((CH + 8, H), jnp.float32),
            pltpu.VMEM((CH, H), jnp.float32),
            pltpu.SemaphoreType.DMA,
        ],
        interpret=interpret,
    )


def _gates(iou, c_red):
    i = jax.nn.sigmoid(iou[:, :H])
    o = jax.nn.sigmoid(iou[:, H:2 * H])
    u = jnp.tanh(iou[:, 2 * H:])
    c = i * u + c_red
    h = o * jnp.tanh(c)
    return h, c


def _subtree_body(e_ref, m_ref, w_ref, b_ref, wa_ref, ba_ref, lw_ref, lb_ref,
                  *out_refs):
    lw = lw_ref[...]
    lb = lb_ref[...]
    iou = (
        jnp.dot(e_ref[...], w_ref[...], preferred_element_type=jnp.float32)
        * m_ref[...]
        + b_ref[...]
    )
    h, c = _gates(iou, 0.0)
    mh = jnp.maximum(h, 0.0)
    lg0 = jnp.dot(h + mh, lw, preferred_element_type=jnp.float32) + lb
    out_refs[0][...] = lg0.reshape(out_refs[0].shape)
    wa = wa_ref[...]
    ba = ba_ref[...]
    s = SUB
    for k in range(1, D - TOPL):
        s //= 2
        hc = h.reshape(s, 2 * H)
        cc = c.reshape(s, 2 * H)
        mc = mh.reshape(s, 2 * H)
        g = jnp.dot(hc, wa, preferred_element_type=jnp.float32) + ba
        f = jax.nn.sigmoid(g[:, :2 * H])
        c_red = f[:, :H] * cc[:, :H] + f[:, H:] * cc[:, H:]
        h, c = _gates(g[:, 2 * H:], c_red)
        mh = jnp.maximum(h, jnp.maximum(mc[:, :H], mc[:, H:]))
        lg = jnp.dot(h + mh, lw, preferred_element_type=jnp.float32) + lb
        out_refs[k][...] = lg.reshape(out_refs[k].shape)
    out_refs[D - TOPL][...] = h.reshape(1, 1, H)
    out_refs[D - TOPL + 1][...] = c.reshape(1, 1, H)
    out_refs[D - TOPL + 2][...] = mh.reshape(1, 1, H)


@functools.cache
def _make_subtree_call(ncls, nsub=NSUB, interpret=False):
    lg_specs, lg_shapes = [], []
    for l in range(D - 1, TOPL - 1, -1):
        bs = 2 ** (l - TOPL)
        lg_specs.append(pl.BlockSpec((1, bs, ncls), lambda g: (g, 0, 0)))
        lg_shapes.append(jax.ShapeDtypeStruct((nsub, bs, ncls), jnp.float32))
    fr_spec = pl.BlockSpec((1, 1, H), lambda g: (g, 0, 0))
    fr_shape = jax.ShapeDtypeStruct((nsub, 1, H), jnp.float32)
    return pl.pallas_call(
        _subtree_body,
        grid=(nsub,),
        in_specs=[
            pl.BlockSpec((SUB, H), lambda g: (g, 0)),
            pl.BlockSpec((SUB, 1), lambda g: (g, 0)),
            pl.BlockSpec((H, 3 * H), lambda g: (0, 0)),
            pl.BlockSpec((1, 3 * H), lambda g: (0, 0)),
            pl.BlockSpec((2 * H, 5 * H), lambda g: (0, 0)),
            pl.BlockSpec((1, 5 * H), lambda g: (0, 0)),
            pl.BlockSpec((H, ncls), lambda g: (0, 0)),
            pl.BlockSpec((1, ncls), lambda g: (0, 0)),
        ],
        out_specs=[*lg_specs, fr_spec, fr_spec, fr_spec],
        out_shape=[*lg_shapes, fr_shape, fr_shape, fr_shape],
        interpret=interpret,
    )


def _top_body(hc_ref, cc_ref, mc_ref, wa_ref, ba_ref, lw_ref, lb_ref, lg_ref):
    lw = lw_ref[...]
    lb = lb_ref[...]
    wa = wa_ref[...]
    ba = ba_ref[...]
    hc, cc, mc = hc_ref[...], cc_ref[...], mc_ref[...]
    for l in range(TOPL - 1, -1, -1):
        s = 2 ** l
        g = jnp.dot(hc, wa, preferred_element_type=jnp.float32) + ba
        f = jax.nn.sigmoid(g[:, :2 * H])
        c_red = f[:, :H] * cc[:, :H] + f[:, H:] * cc[:, H:]
        h, c = _gates(g[:, 2 * H:], c_red)
        mh = jnp.maximum(h, jnp.maximum(mc[:, :H], mc[:, H:]))
        lg_ref[pl.ds(s - 1, s), :] = (
            jnp.dot(h + mh, lw, preferred_element_type=jnp.float32) + lb
        )
        if l > 0:
            hc = h.reshape(s // 2, 2 * H)
            cc = c.reshape(s // 2, 2 * H)
            mc = mh.reshape(s // 2, 2 * H)


@functools.cache
def _make_top_call(ncls, interpret=False):
    s5 = 2 ** TOPL
    return pl.pallas_call(
        _top_body,
        out_shape=jax.ShapeDtypeStruct((s5 - 1, ncls), jnp.float32),
        interpret=interpret,
    )


def _tree_lstm(leaf_embs, maskfs, w_iou_t, b_iou, w_all, b_all, lin_W, lin_b,
               interpret=False):
    ncls = lin_W.shape[0]
    lw = lin_W.T
    lb = lin_b.reshape(1, ncls)
    nhalf = len(leaf_embs)
    houts = []
    for le, mf in zip(leaf_embs, maskfs):
        nsub = le.shape[0] // SUB
        houts.append(_make_subtree_call(ncls, nsub, interpret)(
            le, mf, w_iou_t, b_iou, w_all, b_all, lw, lb
        ))
    lgs = [jnp.concatenate([o.reshape(-1, ncls) for o in
                            [ho[k] for ho in houts]], axis=0)
           for k in range(D - TOPL)]
    h5 = jnp.concatenate([ho[D - TOPL].reshape(-1, H) for ho in houts], axis=0)
    c5 = jnp.concatenate([ho[D - TOPL + 1].reshape(-1, H) for ho in houts], axis=0)
    mh5 = jnp.concatenate([ho[D - TOPL + 2].reshape(-1, H) for ho in houts], axis=0)
    s5 = 2 ** TOPL
    top_lg = _make_top_call(ncls, interpret)(
        h5.reshape(s5 // 2, 2 * H), c5.reshape(s5 // 2, 2 * H),
        mh5.reshape(s5 // 2, 2 * H), w_all, b_all, lw, lb
    )
    return jnp.concatenate([top_lg, *lgs[::-1]], axis=0)


NHALF = 2


def kernel(wordid, mask, emb, W_iou, U_iou, b_iou, U_f_W, U_f_b, lin_W, lin_b):
    leaf_wid = (wordid[LEAF - 1:] * mask[LEAF - 1:]).astype(jnp.int32)
    maskf = mask[LEAF - 1:].astype(jnp.float32).reshape(LEAF, 1)
    hn = LEAF // NHALF
    msk32 = mask[LEAF - 1:].astype(jnp.int32)
    leaf_embs = [_make_sc_gather(hn)(emb, leaf_wid[i * hn:(i + 1) * hn],
                                     msk32[i * hn:(i + 1) * hn])
                 for i in range(NHALF)]
    maskfs = [maskf[i * hn:(i + 1) * hn] for i in range(NHALF)]
    w_all = jnp.concatenate([U_f_W, U_iou], axis=0).T
    b_all = jnp.concatenate([U_f_b, b_iou[0]]).reshape(1, 5 * H)
    return _tree_lstm(leaf_embs, maskfs, W_iou.T, b_iou, w_all, b_all,
                      lin_W, lin_b)
